# baseline (speedup 1.0000x reference)
import functools

import jax
import jax.numpy as jnp
from jax import lax
from jax.experimental import pallas as pl
from jax.experimental.pallas import tpu as pltpu
from jax.experimental.pallas import tpu_sc as plsc

_INPUT_DIM = 1000000
_OUTPUT_DIM = 128
_RANK = 64
_BATCH = 16384

_NC = 2
_NS = 16
_NW = _NC * _NS
_B_PER_W = _BATCH // _NW
_L = 16


def _sc_gather_t(idx, flat):
  mesh = plsc.VectorSubcoreMesh(core_axis_name="c", subcore_axis_name="s")

  @functools.partial(
      pl.kernel,
      out_type=jax.ShapeDtypeStruct((_RANK, _BATCH), jnp.float32),
      mesh=mesh,
      scratch_types=[
          pltpu.VMEM((_B_PER_W,), jnp.int32),
          pltpu.VMEM((_RANK * _B_PER_W,), jnp.int32),
          pltpu.VMEM((_RANK * _B_PER_W,), jnp.float32),
          pltpu.SemaphoreType.DMA,
          pltpu.SemaphoreType.DMA,
      ],
  )
  def gather_kernel(idx_hbm, flat_hbm, out_hbm, idx_v, gidx_v, vals_v,
                    sem, wsem):
    wid = lax.axis_index("s") * _NC + lax.axis_index("c")
    base = wid * _B_PER_W
    pltpu.sync_copy(idx_hbm.at[pl.ds(base, _B_PER_W)], idx_v)

    def build(c, carry):
      g = idx_v[pl.ds(c * _L, _L)]
      for j in range(_RANK):
        gidx_v[pl.ds(j * _B_PER_W + c * _L, _L)] = g + j * _INPUT_DIM
      return carry

    lax.fori_loop(0, _B_PER_W // _L, build, 0)

    pltpu.async_copy(flat_hbm.at[gidx_v], vals_v, sem).wait()

    copies = []
    for j in range(_RANK):
      copies.append(
          pltpu.async_copy(
              vals_v.at[pl.ds(j * _B_PER_W, _B_PER_W)],
              out_hbm.at[j, pl.ds(base, _B_PER_W)],
              wsem,
          ))
    for cp in copies:
      cp.wait()

  return gather_kernel(idx, flat)


def _tc_matmul_sq(x_t, v):
  blk = 2048

  def body(xt_ref, v_ref, o_ref):
    o = lax.dot_general(
        xt_ref[...], v_ref[...],
        (((0,), (1,)), ((), ())),
        preferred_element_type=jnp.float32,
    )
    o_ref[...] = o * o

  return pl.pallas_call(
      body,
      grid=(_BATCH // blk,),
      in_specs=[
          pl.BlockSpec((_RANK, blk), lambda i: (0, i)),
          pl.BlockSpec((_OUTPUT_DIM, _RANK), lambda i: (0, 0)),
      ],
      out_specs=pl.BlockSpec((blk, _OUTPUT_DIM), lambda i: (i, 0)),
      out_shape=jax.ShapeDtypeStruct((_BATCH, _OUTPUT_DIM), jnp.float32),
  )(x_t, v)


def kernel(state_inx, H, V):
  idx = state_inx.astype(jnp.int32)
  flat = jnp.transpose(H).reshape(_INPUT_DIM * _RANK)
  x_t = _sc_gather_t(idx, flat)
  return _tc_matmul_sq(x_t, V)


# speedup vs baseline: 1.0017x; 1.0017x over previous
import functools

import jax
import jax.numpy as jnp
from jax import lax
from jax.experimental import pallas as pl
from jax.experimental.pallas import tpu as pltpu
from jax.experimental.pallas import tpu_sc as plsc

_INPUT_DIM = 1000000
_OUTPUT_DIM = 128
_RANK = 64
_BATCH = 16384

_NC = 2
_NS = 16
_NW = _NC * _NS
_B_PER_W = _BATCH // _NW
_L = 16


def _sc_gather_t(idx, flat):
  mesh = plsc.VectorSubcoreMesh(core_axis_name="c", subcore_axis_name="s")

  @functools.partial(
      pl.kernel,
      out_type=jax.ShapeDtypeStruct((_RANK, _BATCH), jnp.float32),
      mesh=mesh,
      scratch_types=[
          pltpu.VMEM((_B_PER_W,), jnp.int32),
          pltpu.VMEM((_RANK * _B_PER_W,), jnp.float32),
          pltpu.SemaphoreType.DMA,
          pltpu.SemaphoreType.DMA,
      ],
      compiler_params=pltpu.CompilerParams(use_tc_tiling_on_sc=False),
  )
  def gather_kernel(idx_hbm, tabt_hbm, out_hbm, idx_v, vals_v, sem, wsem):
    wid = lax.axis_index("s") * _NC + lax.axis_index("c")
    base = wid * _B_PER_W
    pltpu.sync_copy(idx_hbm.at[pl.ds(base, _B_PER_W)], idx_v)

    gathers = []
    for j in range(_RANK):
      gathers.append(
          pltpu.async_copy(
              tabt_hbm.at[j].at[idx_v],
              vals_v.at[pl.ds(j * _B_PER_W, _B_PER_W)],
              sem,
          ))
    for cp in gathers:
      cp.wait()

    copies = []
    for j in range(_RANK):
      copies.append(
          pltpu.async_copy(
              vals_v.at[pl.ds(j * _B_PER_W, _B_PER_W)],
              out_hbm.at[j, pl.ds(base, _B_PER_W)],
              wsem,
          ))
    for cp in copies:
      cp.wait()

  return gather_kernel(idx, flat)


def _tc_matmul_sq(x_t, v):
  blk = 2048

  def body(xt_ref, v_ref, o_ref):
    o = lax.dot_general(
        xt_ref[...], v_ref[...],
        (((0,), (1,)), ((), ())),
        preferred_element_type=jnp.float32,
    )
    o_ref[...] = o * o

  return pl.pallas_call(
      body,
      grid=(_BATCH // blk,),
      in_specs=[
          pl.BlockSpec((_RANK, blk), lambda i: (0, i)),
          pl.BlockSpec((_OUTPUT_DIM, _RANK), lambda i: (0, 0)),
      ],
      out_specs=pl.BlockSpec((blk, _OUTPUT_DIM), lambda i: (i, 0)),
      out_shape=jax.ShapeDtypeStruct((_BATCH, _OUTPUT_DIM), jnp.float32),
  )(x_t, v)


def kernel(state_inx, H, V):
  idx = state_inx.astype(jnp.int32)
  flat = jnp.transpose(H)
  x_t = _sc_gather_t(idx, flat)
  return _tc_matmul_sq(x_t, V)


# speedup vs baseline: 7.5657x; 7.5532x over previous
import functools

import jax
import jax.numpy as jnp
from jax import lax
from jax.experimental import pallas as pl
from jax.experimental.pallas import tpu as pltpu
from jax.experimental.pallas import tpu_sc as plsc

_INPUT_DIM = 1000000
_OUTPUT_DIM = 128
_RANK = 64
_BATCH = 16384

_NC = 2
_NS = 16
_NW = _NC * _NS
_B_PER_W = _BATCH // _NW
_L = 16


def _sc_gather_t(idx, flat):
  mesh = plsc.VectorSubcoreMesh(core_axis_name="c", subcore_axis_name="s")

  @functools.partial(
      pl.kernel,
      out_type=jax.ShapeDtypeStruct((_RANK, _BATCH), jnp.float32),
      mesh=mesh,
      scratch_types=[
          pltpu.VMEM((_B_PER_W,), jnp.int32),
          pltpu.VMEM((_RANK * _B_PER_W,), jnp.int32),
          pltpu.VMEM((_RANK * _B_PER_W,), jnp.float32),
          pltpu.SemaphoreType.DMA,
          pltpu.SemaphoreType.DMA,
      ],
      compiler_params=pltpu.CompilerParams(use_tc_tiling_on_sc=False),
  )
  def gather_kernel(idx_hbm, tabt_hbm, out_hbm, idx_v, gidx_v, vals_v, sem,
                    wsem):
    wid = lax.axis_index("s") * _NC + lax.axis_index("c")
    base = wid * _B_PER_W
    pltpu.sync_copy(idx_hbm.at[pl.ds(base, _B_PER_W)], idx_v)

    def build(c, carry):
      g = idx_v[pl.ds(c * _L, _L)] * _RANK
      for j in range(_RANK):
        gidx_v[pl.ds(j * _B_PER_W + c * _L, _L)] = g + j
      return carry

    lax.fori_loop(0, _B_PER_W // _L, build, 0)
    pltpu.async_copy(tabt_hbm.at[gidx_v], vals_v, sem).wait()

    copies = []
    for j in range(_RANK):
      copies.append(
          pltpu.async_copy(
              vals_v.at[pl.ds(j * _B_PER_W, _B_PER_W)],
              out_hbm.at[j, pl.ds(base, _B_PER_W)],
              wsem,
          ))
    for cp in copies:
      cp.wait()

  return gather_kernel(idx, flat)


def _tc_matmul_sq(x_t, v):
  blk = 2048

  def body(xt_ref, v_ref, o_ref):
    o = lax.dot_general(
        xt_ref[...], v_ref[...],
        (((0,), (1,)), ((), ())),
        preferred_element_type=jnp.float32,
    )
    o_ref[...] = o * o

  return pl.pallas_call(
      body,
      grid=(_BATCH // blk,),
      in_specs=[
          pl.BlockSpec((_RANK, blk), lambda i: (0, i)),
          pl.BlockSpec((_OUTPUT_DIM, _RANK), lambda i: (0, 0)),
      ],
      out_specs=pl.BlockSpec((blk, _OUTPUT_DIM), lambda i: (i, 0)),
      out_shape=jax.ShapeDtypeStruct((_BATCH, _OUTPUT_DIM), jnp.float32),
  )(x_t, v)


def kernel(state_inx, H, V):
  idx = state_inx.astype(jnp.int32)
  flat = jnp.reshape(H, (_INPUT_DIM * _RANK,))
  x_t = _sc_gather_t(idx, flat)
  return _tc_matmul_sq(x_t, V)


# speedup vs baseline: 8.3059x; 1.0978x over previous
import functools

import jax
import jax.numpy as jnp
from jax import lax
from jax.experimental import pallas as pl
from jax.experimental.pallas import tpu as pltpu
from jax.experimental.pallas import tpu_sc as plsc

_INPUT_DIM = 1000000
_OUTPUT_DIM = 128
_RANK = 64
_BATCH = 16384

_NC = 2
_NS = 16
_NW = _NC * _NS
_B_PER_W = _BATCH // _NW


def _sc_gather(idx, table):
  mesh = plsc.VectorSubcoreMesh(core_axis_name="c", subcore_axis_name="s")
  k = 16

  @functools.partial(
      pl.kernel,
      out_type=jax.ShapeDtypeStruct((_BATCH, _RANK), jnp.float32),
      mesh=mesh,
      scratch_types=[
          pltpu.VMEM((_B_PER_W,), jnp.int32),
          pltpu.SemaphoreType.DMA,
      ],
  )
  def gather_kernel(idx_hbm, table_hbm, out_hbm, idx_v, sem):
    wid = lax.axis_index("s") * _NC + lax.axis_index("c")
    base = wid * _B_PER_W
    pltpu.sync_copy(idx_hbm.at[pl.ds(base, _B_PER_W)], idx_v)

    window = 64
    copies = []
    waited = 0
    for c in range(_B_PER_W // k):
      g = idx_v[pl.ds(c * k, k)]
      for j in range(k):
        i = c * k + j
        r = g[j]
        copies.append(
            pltpu.async_copy(
                table_hbm.at[pl.ds(r, 1)],
                out_hbm.at[pl.ds(base + i, 1)],
                sem,
            ))
        while len(copies) - waited > window:
          copies[waited].wait()
          waited += 1
    while waited < len(copies):
      copies[waited].wait()
      waited += 1

  return gather_kernel(idx, table)


def _tc_matmul_sq(x, v):
  blk = 2048

  def body(x_ref, v_ref, o_ref):
    o = lax.dot_general(
        x_ref[...], v_ref[...],
        (((1,), (1,)), ((), ())),
        preferred_element_type=jnp.float32,
    )
    o_ref[...] = o * o

  return pl.pallas_call(
      body,
      grid=(_BATCH // blk,),
      in_specs=[
          pl.BlockSpec((blk, _RANK), lambda i: (i, 0)),
          pl.BlockSpec((_OUTPUT_DIM, _RANK), lambda i: (0, 0)),
      ],
      out_specs=pl.BlockSpec((blk, _OUTPUT_DIM), lambda i: (i, 0)),
      out_shape=jax.ShapeDtypeStruct((_BATCH, _OUTPUT_DIM), jnp.float32),
  )(x, v)


def kernel(state_inx, H, V):
  idx = state_inx.astype(jnp.int32)
  gathered = _sc_gather(idx, H)
  return _tc_matmul_sq(gathered, V)
